# baseline (speedup 1.0000x reference)
import numpy as np
import jax
import jax.numpy as jnp
from jax.experimental import pallas as pl
from jax.experimental.pallas import tpu as pltpu

N_KPS = 8
N_NODES = 24
NSLOT = 24
ER = 64
NR = 64
EE = 32
ATTN = 32
NEG = -1e30


def _edge_structure():
    srcp = np.zeros((N_NODES * NSLOT,), np.int64)
    dstp = np.zeros((N_NODES * NSLOT,), np.int64)
    madd = np.full((N_NODES, NSLOT, 2), NEG, np.float32)
    for n in range(N_NODES):
        for l in range(NSLOT):
            s = n * NSLOT + l
            srcp[s] = n
            if l == 23:
                dstp[s] = 0
                continue
            j = l + (1 if l >= n else 0)
            dstp[s] = j
            madd[n, l, 0 if j // N_KPS == n // N_KPS else 1] = 0.0
    return srcp, dstp, madd


_SRCP, _DSTP, _MADD = _edge_structure()


def _srnn_kernel(nodes_ref, et_ref, es_ref, madd_ref, seconst_ref, wg_ref,
                 wb_ref, wc_ref, w_out_ref, out_ref,
                 ht_ref, ct_ref, hs_ref, cs_ref, hn_ref, cn_ref):
    B = nodes_ref.shape[0]
    N = nodes_ref.shape[2]
    G = B * N
    E = es_ref.shape[2]

    @pl.when(pl.program_id(0) == 0)
    def _init():
        ht_ref[...] = jnp.zeros_like(ht_ref)
        ct_ref[...] = jnp.zeros_like(ct_ref)
        hs_ref[...] = jnp.zeros_like(hs_ref)
        cs_ref[...] = jnp.zeros_like(cs_ref)
        hn_ref[...] = jnp.zeros_like(hn_ref)
        cn_ref[...] = jnp.zeros_like(cn_ref)

    def sig(x):
        return 0.5 * jnp.tanh(0.5 * x) + 0.5

    def lstm(pre, h, c, whh):
        g = pre + h @ whh
        i = sig(g[:, 0 * ER:1 * ER])
        f = sig(g[:, 1 * ER:2 * ER])
        gg = jnp.tanh(g[:, 2 * ER:3 * ER])
        o = sig(g[:, 3 * ER:4 * ER])
        c2 = f * c + i * gg
        h2 = o * jnp.tanh(c2)
        return h2, c2

    et = et_ref[...].reshape(G, 2)
    te_in = jax.nn.relu(et @ wc_ref[8:10] + wc_ref[16:17])
    h_temp, c_temp = lstm(te_in @ wg_ref[0:32] + wg_ref[320:321],
                          ht_ref[...], ct_ref[...], wg_ref[32:96])
    ht_ref[...] = h_temp
    ct_ref[...] = c_temp

    esr = es_ref[0]
    d2 = jnp.maximum(esr[0:1] * esr[0:1] + esr[1:2] * esr[1:2], 1e-12)
    feat_t = jnp.concatenate(
        [esr * jax.lax.rsqrt(d2), 0.5 * jnp.log(d2)], axis=0)
    se_pre = jax.lax.dot_general(
        feat_t, wc_ref[0:3], (((0,), (0,)), ((), ()))) + seconst_ref[...]
    se_in = jax.nn.relu(se_pre)
    h_spat, c_spat = lstm(se_in @ wg_ref[96:128] + wg_ref[328:329],
                          hs_ref[...], cs_ref[...], wg_ref[128:192])
    hs_ref[...] = h_spat
    cs_ref[...] = c_spat

    q2 = h_temp @ wb_ref[0:64] + wb_ref[128:129]
    k2 = h_spat @ wb_ref[64:128]
    k2 = k2.reshape(G, NSLOT, 2 * ATTN)
    u2 = jnp.tanh(q2[:, None, :] + k2)
    s2 = u2.reshape(E, 2 * ATTN) @ wc_ref[240:304]
    s3 = s2[:, 0:2].reshape(G, NSLOT, 2) + madd_ref[...]
    ex = jnp.exp(s3)
    rden = 1.0 / jnp.sum(ex, axis=1)
    hs3 = h_spat.reshape(G, NSLOT, ER)
    h_ia = jnp.sum(ex[:, :, 0:1] * hs3, axis=1) * rden[:, 0:1]
    h_ea = jnp.sum(ex[:, :, 1:2] * hs3, axis=1) * rden[:, 1:2]

    node_in = jax.nn.relu(nodes_ref[...].reshape(G, 2) @ wc_ref[24:26]
                          + wc_ref[32:33])
    edge_in = jax.nn.relu(h_temp @ wc_ref[40:104] + h_ia @ wc_ref[104:168]
                          + h_ea @ wc_ref[168:232] + wc_ref[232:233])
    pre_n = (node_in @ wg_ref[192:224] + edge_in @ wg_ref[224:256]
             + wg_ref[336:337])
    h_node, c_node = lstm(pre_n, hn_ref[...], cn_ref[...], wg_ref[256:320])
    hn_ref[...] = h_node
    cn_ref[...] = c_node

    res = h_node @ w_out_ref[0:NR] + w_out_ref[NR:NR + 1]
    out_ref[...] = res.reshape(B, 1, N, 5)


def kernel(nodes, edges_temporal, edges_spatial, params):
    p = params
    B, T, N, _ = nodes.shape
    G = B * N
    E = B * N * NSLOT

    es_p = jnp.pad(edges_spatial.reshape(B, T, N, N - 1, 2),
                   ((0, 0), (0, 0), (0, 0), (0, 1), (0, 0)))
    es_t = es_p.transpose(1, 4, 0, 2, 3).reshape(T, 2, E)

    base = np.tile(_MADD, (B, 1, 1))
    shift = jnp.stack([jnp.sum(jnp.abs(p['Ws_intra'])),
                       jnp.sum(jnp.abs(p['Ws_inter']))])
    madd = jnp.asarray(base) - (base == 0.0) * shift[None, None, :]

    kp = p['kp_emb']
    w_se = p['W_se']
    se_const0 = (kp[_SRCP % N_KPS] @ w_se[3:3 + N_KPS]
                 + kp[_DSTP % N_KPS] @ w_se[3 + N_KPS:3 + 2 * N_KPS]
                 + p['b_se'][None, :])
    se_const = jnp.tile(se_const0, (B, 1))

    def r2(x):
        return x.reshape(1, -1)

    def z(r, w):
        return jnp.zeros((r, w), jnp.float32)

    wg = jnp.concatenate([
        p['te_Wih'], p['te_Whh'],
        p['se_Wih'], p['se_Whh'],
        p['nd_Wih'][0:EE], p['nd_Wih'][EE:2 * EE], p['nd_Whh'],
        r2(p['te_bih'] + p['te_bhh']), z(7, 4 * ER),
        r2(p['se_bih'] + p['se_bhh']), z(7, 4 * ER),
        r2(p['nd_bih'] + p['nd_bhh']), z(7, 4 * ER),
    ], axis=0)

    wb = jnp.concatenate([
        jnp.concatenate([p['Wq'], p['Wq']], axis=1),
        jnp.concatenate([p['Wki'], p['Wke']], axis=1),
        jnp.concatenate([p['bq'] + p['bki'], p['bq'] + p['bke']])
        .reshape(1, 2 * ATTN), z(7, 2 * ATTN),
    ], axis=0)

    ws2p = (z(2 * ATTN, EE)
            .at[0:ATTN, 0:1].set(p['Ws_intra'])
            .at[ATTN:2 * ATTN, 1:2].set(p['Ws_inter']))
    wc = jnp.concatenate([
        w_se[0:3], z(5, EE),
        p['W_te'], z(6, EE), r2(p['b_te']), z(7, EE),
        p['W_ne'], z(6, EE), r2(p['b_ne']), z(7, EE),
        p['W_ee'][0:ER], p['W_ee'][ER:2 * ER], p['W_ee'][2 * ER:3 * ER],
        r2(p['b_ee']), z(7, EE),
        ws2p,
    ], axis=0)

    w_out = jnp.concatenate([p['W_out'], r2(p['b_out'])], axis=0)

    weights = (madd, se_const, wg, wb, wc, w_out)

    def full(x):
        nd = x.ndim
        return pl.BlockSpec(x.shape, lambda t, _n=nd: (0,) * _n)

    in_specs = [
        pl.BlockSpec((B, 1, N, 2), lambda t: (0, t, 0, 0)),
        pl.BlockSpec((B, 1, N, 2), lambda t: (0, t, 0, 0)),
        pl.BlockSpec((1, 2, E), lambda t: (t, 0, 0)),
    ] + [full(w) for w in weights]

    out = pl.pallas_call(
        _srnn_kernel,
        grid=(T,),
        in_specs=in_specs,
        out_specs=pl.BlockSpec((B, 1, N, 5), lambda t: (0, t, 0, 0)),
        out_shape=jax.ShapeDtypeStruct((B, T, N, 5), jnp.float32),
        scratch_shapes=[
            pltpu.VMEM((G, ER), jnp.float32),
            pltpu.VMEM((G, ER), jnp.float32),
            pltpu.VMEM((E, ER), jnp.float32),
            pltpu.VMEM((E, ER), jnp.float32),
            pltpu.VMEM((G, NR), jnp.float32),
            pltpu.VMEM((G, NR), jnp.float32),
        ],
        compiler_params=pltpu.CompilerParams(
            dimension_semantics=("arbitrary",)),
    )(nodes, edges_temporal, es_t, *weights)
    return out


# speedup vs baseline: 1.0330x; 1.0330x over previous
import numpy as np
import jax
import jax.numpy as jnp
from jax.experimental import pallas as pl
from jax.experimental.pallas import tpu as pltpu

N_KPS = 8
N_NODES = 24
ER = 64
NR = 64
EE = 32
ATTN = 32
NEG = -1e30


def _edge_structure():
    gather_idx = np.zeros((N_NODES * N_NODES,), np.int32)
    src, dst = [], []
    e = 0
    for i in range(N_NODES):
        for j in range(N_NODES):
            if i == j:
                continue
            gather_idx[i * N_NODES + j] = e
            src.append(i)
            dst.append(j)
            e += 1
    madd = np.full((N_NODES, N_NODES, 2), NEG, np.float32)
    for n in range(N_NODES):
        for j in range(N_NODES):
            if j == n:
                continue
            if j // N_KPS == n // N_KPS:
                madd[n, j, 0] = 0.0
            else:
                madd[n, j, 1] = 0.0
    return np.array(src), np.array(dst), gather_idx, madd


_SRC, _DST, _GATHER, _MADD = _edge_structure()


def _srnn_kernel(nodes_ref, et_ref, es_ref, madd_ref, seconst_ref, w_te_ref,
                 b_te_ref, te_wih_ref, te_whh_ref, te_b_ref, w3_ref,
                 se_wih_ref, se_whh_ref, se_b_ref, wq2_ref,
                 wk2_ref, bqk2_ref, ws2_ref, w_ne_ref, b_ne_ref, w_ee_t_ref,
                 w_ee_i_ref, w_ee_e_ref, b_ee_ref, nd_wih_n_ref, nd_wih_e_ref,
                 nd_whh_ref, nd_b_ref, w_out_ref, b_out_ref, out_ref,
                 ht_ref, ct_ref, hs_ref, cs_ref, hn_ref, cn_ref):
    B = nodes_ref.shape[0]
    N = nodes_ref.shape[2]
    G = B * N
    E = es_ref.shape[2]

    @pl.when(pl.program_id(0) == 0)
    def _init():
        ht_ref[...] = jnp.zeros_like(ht_ref)
        ct_ref[...] = jnp.zeros_like(ct_ref)
        hs_ref[...] = jnp.zeros_like(hs_ref)
        cs_ref[...] = jnp.zeros_like(cs_ref)
        hn_ref[...] = jnp.zeros_like(hn_ref)
        cn_ref[...] = jnp.zeros_like(cn_ref)

    def sig(x):
        return 0.5 * jnp.tanh(0.5 * x) + 0.5

    def lstm(pre, h, c, whh_ref):
        g = pre + h @ whh_ref[...]
        i = sig(g[:, 0 * ER:1 * ER])
        f = sig(g[:, 1 * ER:2 * ER])
        gg = jnp.tanh(g[:, 2 * ER:3 * ER])
        o = sig(g[:, 3 * ER:4 * ER])
        c2 = f * c + i * gg
        h2 = o * jnp.tanh(c2)
        return h2, c2

    et = et_ref[...].reshape(G, 2)
    te_in = jax.nn.relu(et @ w_te_ref[...] + b_te_ref[...])
    h_temp, c_temp = lstm(te_in @ te_wih_ref[...] + te_b_ref[...],
                          ht_ref[...], ct_ref[...], te_whh_ref)
    ht_ref[...] = h_temp
    ct_ref[...] = c_temp

    esr = es_ref[0]
    d2 = jnp.maximum(esr[0:1] * esr[0:1] + esr[1:2] * esr[1:2], 1e-12)
    feat_t = jnp.concatenate(
        [esr * jax.lax.rsqrt(d2), 0.5 * jnp.log(d2)], axis=0)
    se_pre = jax.lax.dot_general(
        feat_t, w3_ref[...], (((0,), (0,)), ((), ()))) + seconst_ref[...]
    se_in = jax.nn.relu(se_pre)
    h_spat, c_spat = lstm(se_in @ se_wih_ref[...] + se_b_ref[...],
                          hs_ref[...], cs_ref[...], se_whh_ref)
    hs_ref[...] = h_spat
    cs_ref[...] = c_spat

    q2 = h_temp @ wq2_ref[...] + bqk2_ref[...]
    k2 = h_spat @ wk2_ref[...]
    k2 = k2.reshape(G, N_NODES, 2 * ATTN)
    u2 = jnp.tanh(q2[:, None, :] + k2)
    s2 = u2.reshape(E, 2 * ATTN) @ ws2_ref[...]
    s3 = s2.reshape(G, N_NODES, 2) + madd_ref[...]
    ex = jnp.exp(s3)
    rden = 1.0 / jnp.sum(ex, axis=1)
    hs3 = h_spat.reshape(G, N_NODES, ER)
    h_ia = jnp.sum(ex[:, :, 0:1] * hs3, axis=1) * rden[:, 0:1]
    h_ea = jnp.sum(ex[:, :, 1:2] * hs3, axis=1) * rden[:, 1:2]

    node_in = jax.nn.relu(nodes_ref[...].reshape(G, 2) @ w_ne_ref[...]
                          + b_ne_ref[...])
    edge_in = jax.nn.relu(h_temp @ w_ee_t_ref[...] + h_ia @ w_ee_i_ref[...]
                          + h_ea @ w_ee_e_ref[...] + b_ee_ref[...])
    pre_n = (node_in @ nd_wih_n_ref[...] + edge_in @ nd_wih_e_ref[...]
             + nd_b_ref[...])
    h_node, c_node = lstm(pre_n, hn_ref[...], cn_ref[...], nd_whh_ref)
    hn_ref[...] = h_node
    cn_ref[...] = c_node

    res = h_node @ w_out_ref[...] + b_out_ref[...]
    out_ref[...] = res.reshape(B, 1, N, 5)


def kernel(nodes, edges_temporal, edges_spatial, params):
    p = params
    B, T, N, _ = nodes.shape
    G = B * N
    E = B * N * N

    es_p = jnp.take(edges_spatial, jnp.asarray(_GATHER), axis=2)
    es_t = es_p.transpose(1, 3, 0, 2).reshape(T, 2, E)
    base = np.tile(_MADD, (B, 1, 1))
    shift = jnp.stack([jnp.sum(jnp.abs(p['Ws_intra'])),
                       jnp.sum(jnp.abs(p['Ws_inter']))])
    madd = jnp.asarray(base) - (base == 0.0) * shift[None, None, :]

    kp = p['kp_emb']
    w_se = p['W_se']
    se_const0 = (kp[_SRC % N_KPS] @ w_se[3:3 + N_KPS]
                 + kp[_DST % N_KPS] @ w_se[3 + N_KPS:3 + 2 * N_KPS]
                 + p['b_se'][None, :])
    se_const = jnp.tile(se_const0[jnp.asarray(_GATHER)], (B, 1))

    def r2(x):
        return x.reshape(1, -1)

    wq2 = jnp.concatenate([p['Wq'], p['Wq']], axis=1)
    wk2 = jnp.concatenate([p['Wki'], p['Wke']], axis=1)
    bqk2 = (jnp.concatenate([p['bq'] + p['bki'], p['bq'] + p['bke']])
            .reshape(1, 2 * ATTN))
    ws2 = jnp.zeros((2 * ATTN, 2), jnp.float32)
    ws2 = ws2.at[:ATTN, 0:1].set(p['Ws_intra']).at[ATTN:, 1:2].set(p['Ws_inter'])

    weights = (
        madd, se_const,
        p['W_te'], r2(p['b_te']),
        p['te_Wih'], p['te_Whh'], r2(p['te_bih'] + p['te_bhh']),
        w_se[0:3],
        p['se_Wih'], p['se_Whh'], r2(p['se_bih'] + p['se_bhh']),
        wq2, wk2, bqk2, ws2,
        p['W_ne'], r2(p['b_ne']),
        p['W_ee'][0:ER], p['W_ee'][ER:2 * ER], p['W_ee'][2 * ER:3 * ER],
        r2(p['b_ee']),
        p['nd_Wih'][0:EE], p['nd_Wih'][EE:2 * EE],
        p['nd_Whh'], r2(p['nd_bih'] + p['nd_bhh']),
        p['W_out'], r2(p['b_out']),
    )

    def full(x):
        nd = x.ndim
        return pl.BlockSpec(x.shape, lambda t, _n=nd: (0,) * _n)

    in_specs = [
        pl.BlockSpec((B, 1, N, 2), lambda t: (0, t, 0, 0)),
        pl.BlockSpec((B, 1, N, 2), lambda t: (0, t, 0, 0)),
        pl.BlockSpec((1, 2, E), lambda t: (t, 0, 0)),
    ] + [full(w) for w in weights]

    out = pl.pallas_call(
        _srnn_kernel,
        grid=(T,),
        in_specs=in_specs,
        out_specs=pl.BlockSpec((B, 1, N, 5), lambda t: (0, t, 0, 0)),
        out_shape=jax.ShapeDtypeStruct((B, T, N, 5), jnp.float32),
        scratch_shapes=[
            pltpu.VMEM((G, ER), jnp.float32),
            pltpu.VMEM((G, ER), jnp.float32),
            pltpu.VMEM((E, ER), jnp.float32),
            pltpu.VMEM((E, ER), jnp.float32),
            pltpu.VMEM((G, NR), jnp.float32),
            pltpu.VMEM((G, NR), jnp.float32),
        ],
        compiler_params=pltpu.CompilerParams(
            dimension_semantics=("arbitrary",)),
    )(nodes, edges_temporal, es_t, *weights)
    return out
